# Initial kernel scaffold; baseline (speedup 1.0000x reference)
import functools

import jax
import jax.numpy as jnp
from jax import lax
from jax.experimental import pallas as pl
from jax.experimental.pallas import tpu as pltpu
from jax.experimental.pallas import tpu_sc as plsc

NC = 2
NS = 16
NW = NC * NS
B = 128


def _sc_mesh():
    return plsc.VectorSubcoreMesh(
        core_axis_name="c", subcore_axis_name="s", num_cores=NC, num_subcores=NS
    )


@functools.lru_cache(maxsize=None)
def _make_deg_kernel(npad, ept):
    rpt = npad // NS
    nchunks = ept // B

    @functools.partial(
        pl.kernel,
        out_type=jax.ShapeDtypeStruct((NC, npad, 16), jnp.float32),
        mesh=_sc_mesh(),
        scratch_types=[
            pltpu.VMEM((B,), jnp.int32),
            pltpu.VMEM((B, 16), jnp.float32),
            pltpu.VMEM_SHARED((npad, 16), jnp.float32),
        ],
    )
    def deg_kernel(dst_hbm, ones_hbm, zeros_hbm, out_hbm, dst_v, ones_v, acc_sh):
        c = lax.axis_index("c")
        s = lax.axis_index("s")
        wid = s * NC + c
        row0 = s * rpt
        pltpu.sync_copy(zeros_hbm, acc_sh.at[pl.ds(row0, rpt)])
        pltpu.sync_copy(ones_hbm, ones_v)
        plsc.subcore_barrier()
        base = wid * ept

        def body(i, carry):
            pltpu.sync_copy(dst_hbm.at[pl.ds(base + i * B, B)], dst_v)
            pltpu.sync_copy(ones_v, acc_sh.at[dst_v], add=True)
            return carry

        lax.fori_loop(0, nchunks, body, 0)
        plsc.subcore_barrier()
        pltpu.sync_copy(acc_sh.at[pl.ds(row0, rpt)], out_hbm.at[c, pl.ds(row0, rpt)])

    return deg_kernel


@functools.lru_cache(maxsize=None)
def _make_agg_kernel(npad, ept):
    rpt = npad // NS
    nchunks = ept // B

    @functools.partial(
        pl.kernel,
        out_type=jax.ShapeDtypeStruct((NC, npad, 128), jnp.float32),
        mesh=_sc_mesh(),
        scratch_types=[
            pltpu.VMEM((B,), jnp.int32),
            pltpu.VMEM((B,), jnp.int32),
            pltpu.VMEM((B, 128), jnp.float32),
            pltpu.VMEM_SHARED((npad, 128), jnp.float32),
            pltpu.SemaphoreType.DMA,
        ],
    )
    def agg_kernel(y_hbm, src_hbm, dst_hbm, zeros_hbm, out_hbm,
                   src_v, dst_v, rows_v, acc_sh, sem):
        c = lax.axis_index("c")
        s = lax.axis_index("s")
        wid = s * NC + c
        row0 = s * rpt
        pltpu.sync_copy(zeros_hbm, acc_sh.at[pl.ds(row0, rpt)])
        plsc.subcore_barrier()
        base = wid * ept

        def body(i, carry):
            off = base + i * B
            pltpu.sync_copy(src_hbm.at[pl.ds(off, B)], src_v)
            pltpu.sync_copy(dst_hbm.at[pl.ds(off, B)], dst_v)
            pltpu.async_copy(y_hbm.at[src_v], rows_v, sem).wait()
            pltpu.sync_copy(rows_v, acc_sh.at[dst_v], add=True)
            return carry

        lax.fori_loop(0, nchunks, body, 0)
        plsc.subcore_barrier()
        pltpu.sync_copy(acc_sh.at[pl.ds(row0, rpt)], out_hbm.at[c, pl.ds(row0, rpt)])

    return agg_kernel


def _tc1_body(x_ref, degp_ref, w1_ref, y_ref, dinv_ref):
    dp = degp_ref[...]
    deg = dp[0, :, 0:1] + dp[1, :, 0:1] + 1.0
    dinv = lax.rsqrt(deg)
    xw = jnp.dot(x_ref[...], w1_ref[...], preferred_element_type=jnp.float32)
    y_ref[...] = xw * dinv
    dinv_ref[...] = dinv


def _tc2_body(y1_ref, p_ref, dinv_ref, b1_ref, w2_ref, y2_ref):
    pr = p_ref[...]
    dinv = dinv_ref[...]
    h = jnp.maximum(dinv * (pr[0] + pr[1] + y1_ref[...]) + b1_ref[...], 0.0)
    y2_ref[...] = jnp.dot(h, w2_ref[...], preferred_element_type=jnp.float32) * dinv


def _tc3_body(y2_ref, q_ref, dinv_ref, b2_ref, wout_ref, bout_ref, o_ref):
    qr = q_ref[...]
    h = jnp.maximum(dinv_ref[...] * (qr[0] + qr[1] + y2_ref[...]) + b2_ref[...], 0.0)
    o_ref[...] = jnp.dot(h, wout_ref[...], preferred_element_type=jnp.float32) + bout_ref[...]


def kernel(x, edge_index, W1, b1, W2, b2, Wout, bout):
    n, d = x.shape
    h_dim = W1.shape[1]
    c_dim = Wout.shape[1]
    e = edge_index.shape[1]

    npad = -(-(n + 1) // NW) * NW
    nchunks = -(-e // (NW * B))
    ep = nchunks * NW * B
    ept = nchunks * B
    rpt = npad // NS

    src = edge_index[0]
    dst = edge_index[1]
    pad = ep - e
    if pad:
        src = jnp.concatenate([src, jnp.zeros((pad,), src.dtype)])
        dst = jnp.concatenate([dst, jnp.full((pad,), n, dst.dtype)])
    src = src.astype(jnp.int32)
    dst = dst.astype(jnp.int32)

    ones16 = jnp.ones((B, 16), jnp.float32)
    zeros16 = jnp.zeros((rpt, 16), jnp.float32)
    zeros128 = jnp.zeros((rpt, 128), jnp.float32)

    deg_k = _make_deg_kernel(npad, ept)
    agg_k = _make_agg_kernel(npad, ept)

    degp = deg_k(dst, ones16, zeros16)

    r = 2000
    grid = (n // r,)
    bcast = lambda i: (0, 0)
    row_im = lambda i: (i, 0)
    part_im = lambda i: (0, i, 0)

    y1, dinv = pl.pallas_call(
        _tc1_body,
        grid=grid,
        in_specs=[
            pl.BlockSpec((r, d), row_im),
            pl.BlockSpec((NC, r, 16), part_im),
            pl.BlockSpec((d, h_dim), bcast),
        ],
        out_specs=[
            pl.BlockSpec((r, h_dim), row_im),
            pl.BlockSpec((r, 1), row_im),
        ],
        out_shape=[
            jax.ShapeDtypeStruct((n, h_dim), jnp.float32),
            jax.ShapeDtypeStruct((n, 1), jnp.float32),
        ],
    )(x, degp, W1)

    p1 = agg_k(y1, src, dst, zeros128)

    y2 = pl.pallas_call(
        _tc2_body,
        grid=grid,
        in_specs=[
            pl.BlockSpec((r, h_dim), row_im),
            pl.BlockSpec((NC, r, h_dim), part_im),
            pl.BlockSpec((r, 1), row_im),
            pl.BlockSpec((1, h_dim), bcast),
            pl.BlockSpec((h_dim, h_dim), bcast),
        ],
        out_specs=pl.BlockSpec((r, h_dim), row_im),
        out_shape=jax.ShapeDtypeStruct((n, h_dim), jnp.float32),
    )(y1, p1, dinv, b1.reshape(1, -1), W2)

    p2 = agg_k(y2, src, dst, zeros128)

    out = pl.pallas_call(
        _tc3_body,
        grid=grid,
        in_specs=[
            pl.BlockSpec((r, h_dim), row_im),
            pl.BlockSpec((NC, r, h_dim), part_im),
            pl.BlockSpec((r, 1), row_im),
            pl.BlockSpec((1, h_dim), bcast),
            pl.BlockSpec((h_dim, c_dim), bcast),
            pl.BlockSpec((1, c_dim), bcast),
        ],
        out_specs=pl.BlockSpec((r, c_dim), row_im),
        out_shape=jax.ShapeDtypeStruct((n, c_dim), jnp.float32),
    )(y2, p2, dinv, b2.reshape(1, -1), Wout, bout.reshape(1, -1))

    return out


# speedup vs baseline: 10.4400x; 10.4400x over previous
import functools

import jax
import jax.numpy as jnp
from jax import lax
from jax.experimental import pallas as pl
from jax.experimental.pallas import tpu as pltpu
from jax.experimental.pallas import tpu_sc as plsc

NC = 2
NS = 16
NW = NC * NS
B = 128


def _sc_mesh():
    return plsc.VectorSubcoreMesh(
        core_axis_name="c", subcore_axis_name="s", num_cores=NC, num_subcores=NS
    )


@functools.lru_cache(maxsize=None)
def _make_deg_kernel(npad, ept):
    rpt = npad // NS
    nchunks = ept // B

    @functools.partial(
        pl.kernel,
        out_type=jax.ShapeDtypeStruct((NC, npad, 16), jnp.float32),
        mesh=_sc_mesh(),
        scratch_types=[
            pltpu.VMEM((B,), jnp.int32),
            pltpu.VMEM((B, 16), jnp.float32),
            pltpu.VMEM_SHARED((npad, 16), jnp.float32),
        ],
    )
    def deg_kernel(dst_hbm, ones_hbm, zeros_hbm, out_hbm, dst_v, ones_v, acc_sh):
        c = lax.axis_index("c")
        s = lax.axis_index("s")
        wid = s * NC + c
        row0 = s * rpt
        pltpu.sync_copy(zeros_hbm, acc_sh.at[pl.ds(row0, rpt)])
        pltpu.sync_copy(ones_hbm, ones_v)
        plsc.subcore_barrier()
        base = wid * ept

        def body(i, carry):
            pltpu.sync_copy(dst_hbm.at[pl.ds(base + i * B, B)], dst_v)
            pltpu.sync_copy(ones_v, acc_sh.at[dst_v], add=True)
            return carry

        lax.fori_loop(0, nchunks, body, 0)
        plsc.subcore_barrier()
        pltpu.sync_copy(acc_sh.at[pl.ds(row0, rpt)], out_hbm.at[c, pl.ds(row0, rpt)])

    return deg_kernel


@functools.lru_cache(maxsize=None)
def _make_agg_kernel(npad, ept):
    rpt = npad // NS
    nchunks = ept // B

    @functools.partial(
        pl.kernel,
        out_type=jax.ShapeDtypeStruct((NC, npad, 128), jnp.float32),
        mesh=_sc_mesh(),
        scratch_types=[
            pltpu.VMEM((B,), jnp.int32),
            pltpu.VMEM((B,), jnp.int32),
            pltpu.VMEM((B, 128), jnp.float32),
            pltpu.VMEM_SHARED((npad, 128), jnp.float32),
            pltpu.SemaphoreType.DMA,
        ],
    )
    def agg_kernel(y_hbm, src_hbm, dst_hbm, zeros_hbm, out_hbm,
                   src_v, dst_v, rows_v, acc_sh, sem):
        c = lax.axis_index("c")
        s = lax.axis_index("s")
        wid = s * NC + c
        row0 = s * rpt
        pltpu.sync_copy(zeros_hbm, acc_sh.at[pl.ds(row0, rpt)])
        plsc.subcore_barrier()
        base = wid * ept

        def body(i, carry):
            off = base + i * B
            pltpu.sync_copy(src_hbm.at[pl.ds(off, B)], src_v)
            pltpu.sync_copy(dst_hbm.at[pl.ds(off, B)], dst_v)
            pltpu.async_copy(y_hbm.at[src_v], rows_v, sem).wait()
            pltpu.sync_copy(rows_v, acc_sh.at[dst_v], add=True)
            return carry

        lax.fori_loop(0, nchunks, body, 0)
        plsc.subcore_barrier()
        pltpu.sync_copy(acc_sh.at[pl.ds(row0, rpt)], out_hbm.at[c, pl.ds(row0, rpt)])

    return agg_kernel


def _tc1_body(x_ref, degp_ref, w1_ref, y_ref, dinv_ref):
    dp = degp_ref[...]
    deg = dp[0, :, 0:1] + dp[1, :, 0:1] + 1.0
    dinv = lax.rsqrt(deg)
    xw = jnp.dot(x_ref[...], w1_ref[...], preferred_element_type=jnp.float32)
    y_ref[...] = xw * dinv
    dinv_ref[...] = dinv


def _tc2_body(y1_ref, p_ref, dinv_ref, b1_ref, w2_ref, y2_ref):
    pr = p_ref[...]
    dinv = dinv_ref[...]
    h = jnp.maximum(dinv * (pr[0] + pr[1] + y1_ref[...]) + b1_ref[...], 0.0)
    y2_ref[...] = jnp.dot(h, w2_ref[...], preferred_element_type=jnp.float32) * dinv


def _tc3_body(y2_ref, q_ref, dinv_ref, b2_ref, wout_ref, bout_ref, o_ref):
    qr = q_ref[...]
    h = jnp.maximum(dinv_ref[...] * (qr[0] + qr[1] + y2_ref[...]) + b2_ref[...], 0.0)
    o_ref[...] = jnp.dot(h, wout_ref[...], preferred_element_type=jnp.float32) + bout_ref[...]


def kernel(x, edge_index, W1, b1, W2, b2, Wout, bout):
    n, d = x.shape
    h_dim = W1.shape[1]
    c_dim = Wout.shape[1]
    e = edge_index.shape[1]

    npad = -(-(n + 1) // (NS * 8)) * (NS * 8)
    nchunks = -(-e // (NW * B))
    ep = nchunks * NW * B
    ept = nchunks * B
    rpt = npad // NS

    src = edge_index[0]
    dst = edge_index[1]
    pad = ep - e
    if pad:
        src = jnp.concatenate([src, jnp.zeros((pad,), src.dtype)])
        dst = jnp.concatenate([dst, jnp.full((pad,), n, dst.dtype)])
    src = src.astype(jnp.int32)
    dst = dst.astype(jnp.int32)

    ones16 = jnp.ones((B, 16), jnp.float32)
    zeros16 = jnp.zeros((rpt, 16), jnp.float32)
    zeros128 = jnp.zeros((rpt, 128), jnp.float32)

    deg_k = _make_deg_kernel(npad, ept)
    agg_k = _make_agg_kernel(npad, ept)

    degp = deg_k(dst, ones16, zeros16)

    r = 2000
    grid = (n // r,)
    bcast = lambda i: (0, 0)
    row_im = lambda i: (i, 0)
    part_im = lambda i: (0, i, 0)

    y1, dinv = pl.pallas_call(
        _tc1_body,
        grid=grid,
        in_specs=[
            pl.BlockSpec((r, d), row_im),
            pl.BlockSpec((NC, r, 16), part_im),
            pl.BlockSpec((d, h_dim), bcast),
        ],
        out_specs=[
            pl.BlockSpec((r, h_dim), row_im),
            pl.BlockSpec((r, 1), row_im),
        ],
        out_shape=[
            jax.ShapeDtypeStruct((n, h_dim), jnp.float32),
            jax.ShapeDtypeStruct((n, 1), jnp.float32),
        ],
    )(x, degp, W1)

    p1 = agg_k(y1, src, dst, zeros128)

    y2 = pl.pallas_call(
        _tc2_body,
        grid=grid,
        in_specs=[
            pl.BlockSpec((r, h_dim), row_im),
            pl.BlockSpec((NC, r, h_dim), part_im),
            pl.BlockSpec((r, 1), row_im),
            pl.BlockSpec((1, h_dim), bcast),
            pl.BlockSpec((h_dim, h_dim), bcast),
        ],
        out_specs=pl.BlockSpec((r, h_dim), row_im),
        out_shape=jax.ShapeDtypeStruct((n, h_dim), jnp.float32),
    )(y1, p1, dinv, b1.reshape(1, -1), W2)

    p2 = agg_k(y2, src, dst, zeros128)

    out = pl.pallas_call(
        _tc3_body,
        grid=grid,
        in_specs=[
            pl.BlockSpec((r, h_dim), row_im),
            pl.BlockSpec((NC, r, h_dim), part_im),
            pl.BlockSpec((r, 1), row_im),
            pl.BlockSpec((1, h_dim), bcast),
            pl.BlockSpec((h_dim, c_dim), bcast),
            pl.BlockSpec((1, c_dim), bcast),
        ],
        out_specs=pl.BlockSpec((r, c_dim), row_im),
        out_shape=jax.ShapeDtypeStruct((n, c_dim), jnp.float32),
    )(y2, p2, dinv, b2.reshape(1, -1), Wout, bout.reshape(1, -1))

    return out
